# baseline (speedup 1.0000x reference)
import jax
import jax.numpy as jnp
from jax import lax
from jax.experimental import pallas as pl
from jax.experimental.pallas import tpu as pltpu
from jax.experimental.pallas import tpu_sc as plsc

VOCAB = 1_000_000
D = 64
B = 16384
NC = 2
NS = 16
NW = NC * NS
BPW = B // NW
L = 16
NG = BPW // L
NQ = 4
AQ = 2
SW = 16


def _sc_body(w1_hbm, w2_hbm, W1_hbm, W2_hbm, out_hbm,
             idx1_v, idx2_v, s1_v, s2_v, dots_v, sem):
    wid = lax.axis_index("s") * NC + lax.axis_index("c")
    base = wid * BPW

    pltpu.sync_copy(w1_hbm.at[pl.ds(base, BPW)], idx1_v)
    pltpu.sync_copy(w2_hbm.at[pl.ds(base, BPW)], idx2_v)

    lanes = lax.iota(jnp.int32, L)

    def group(g, _):
        gsl = pl.ds(g * L, L)
        idv1 = idx1_v[gsl]
        idv2 = idx2_v[gsl]
        o1 = idv1 & (SW - 1)
        o2 = idv2 & (SW - 1)

        for j in range(L):
            st1 = (idv1[j] >> 4) * SW
            st2 = (idv2[j] >> 4) * SW
            for q in range(NQ):
                asl = pl.ds(q * AQ, AQ)
                csl = pl.ds(q * SW, SW)
                pltpu.async_copy(W1_hbm.at[asl, :, pl.ds(st1, SW)],
                                 s1_v.at[j, :, :, csl], sem)
                pltpu.async_copy(W2_hbm.at[asl, :, pl.ds(st2, SW)],
                                 s2_v.at[j, :, :, csl], sem)
        for j in range(L):
            for q in range(NQ):
                csl = pl.ds(q * SW, SW)
                pltpu.make_async_copy(W1_hbm.at[pl.ds(0, AQ), :,
                                                pl.ds(0, SW)],
                                      s1_v.at[j, :, :, csl], sem).wait()
                pltpu.make_async_copy(W1_hbm.at[pl.ds(0, AQ), :,
                                                pl.ds(0, SW)],
                                      s2_v.at[j, :, :, csl], sem).wait()

        acc = jnp.zeros((L,), jnp.float32)
        for s in range(D):
            a = s // 8
            av = jnp.full((L,), a % AQ, jnp.int32)
            bv = jnp.full((L,), s % 8, jnp.int32)
            c1 = o1 + (a // AQ) * SW
            c2 = o2 + (a // AQ) * SW
            va = plsc.load_gather(s1_v, [lanes, av, bv, c1])
            vb = plsc.load_gather(s2_v, [lanes, av, bv, c2])
            acc = acc + va * vb
        dots_v[gsl] = acc
        return 0

    lax.fori_loop(0, NG, group, 0)

    pltpu.sync_copy(dots_v, out_hbm.at[pl.ds(base, BPW)])


def _sc_dots(w1, w2, W1t3, W2t3):
    mesh = plsc.VectorSubcoreMesh(core_axis_name="c", subcore_axis_name="s")
    return pl.kernel(
        _sc_body,
        out_type=jax.ShapeDtypeStruct((B,), jnp.float32),
        mesh=mesh,
        compiler_params=pltpu.CompilerParams(needs_layout_passes=False),
        scratch_types=[
            pltpu.VMEM((BPW,), jnp.int32),
            pltpu.VMEM((BPW,), jnp.int32),
            pltpu.VMEM((L, AQ, 8, 128), jnp.float32),
            pltpu.VMEM((L, AQ, 8, 128), jnp.float32),
            pltpu.VMEM((BPW,), jnp.float32),
            pltpu.SemaphoreType.DMA,
        ],
    )(w1, w2, W1t3, W2t3)


def _loss_body(dot_ref, lbl_ref, out_ref):
    x = dot_ref[...]
    l = lbl_ref[...]
    sp = jnp.maximum(-x, 0.0) + jnp.log1p(jnp.exp(-jnp.abs(x)))
    out_ref[0, 0] = jnp.sum(l * sp) * (1.0 / B)


def _tc_loss(dots2d, lbl2d):
    return pl.pallas_call(
        _loss_body,
        out_shape=jax.ShapeDtypeStruct((1, 1), jnp.float32),
        out_specs=pl.BlockSpec(memory_space=pltpu.SMEM),
    )(dots2d, lbl2d)


def kernel(word1, word2, label, W1, W2):
    W1t3 = W1.T.reshape(8, 8, VOCAB)
    W2t3 = W2.T.reshape(8, 8, VOCAB)
    dots = _sc_dots(word1.astype(jnp.int32), word2.astype(jnp.int32),
                    W1t3, W2t3)
    loss = _tc_loss(dots.reshape(128, 128),
                    label.astype(jnp.float32).reshape(128, 128))
    return loss[0, 0]


# speedup vs baseline: 1.5779x; 1.5779x over previous
import jax
import jax.numpy as jnp
from jax import lax
from jax.experimental import pallas as pl
from jax.experimental.pallas import tpu as pltpu
from jax.experimental.pallas import tpu_sc as plsc

VOCAB = 1_000_000
D = 64
B = 16384
NC = 2
NS = 16
NW = NC * NS
BPW = B // NW
L = 16
NG = BPW // L
NQ = 4
AQ = 2
SW = 16


def _sc_body(w1_hbm, w2_hbm, W1_hbm, W2_hbm, out_hbm,
             idx1_v, idx2_v, s1_v, s2_v, dots_v, sem):
    wid = lax.axis_index("s") * NC + lax.axis_index("c")
    base = wid * BPW

    pltpu.sync_copy(w1_hbm.at[pl.ds(base, BPW)], idx1_v)
    pltpu.sync_copy(w2_hbm.at[pl.ds(base, BPW)], idx2_v)

    lanes = lax.iota(jnp.int32, L)

    def group(g, _):
        gsl = pl.ds(g * L, L)
        idv1 = idx1_v[gsl]
        idv2 = idx2_v[gsl]
        o1 = idv1 & (SW - 1)
        o2 = idv2 & (SW - 1)

        def one_pass(q, acc):
            asl = pl.ds(q * AQ, AQ)
            for j in range(L):
                st1 = (idv1[j] >> 4) * SW
                st2 = (idv2[j] >> 4) * SW
                pltpu.async_copy(W1_hbm.at[asl, :, pl.ds(st1, SW)],
                                 s1_v.at[j, :, :, pl.ds(0, SW)], sem)
                pltpu.async_copy(W2_hbm.at[asl, :, pl.ds(st2, SW)],
                                 s2_v.at[j, :, :, pl.ds(0, SW)], sem)
            for j in range(L):
                pltpu.make_async_copy(W1_hbm.at[pl.ds(0, AQ), :,
                                                pl.ds(0, SW)],
                                      s1_v.at[j, :, :, pl.ds(0, SW)],
                                      sem).wait()
                pltpu.make_async_copy(W1_hbm.at[pl.ds(0, AQ), :,
                                                pl.ds(0, SW)],
                                      s2_v.at[j, :, :, pl.ds(0, SW)],
                                      sem).wait()

            for s in range(AQ * 8):
                av = jnp.full((L,), s // 8, jnp.int32)
                bv = jnp.full((L,), s % 8, jnp.int32)
                va = plsc.load_gather(s1_v, [lanes, av, bv, o1])
                vb = plsc.load_gather(s2_v, [lanes, av, bv, o2])
                acc = acc + va * vb
            return acc

        acc = lax.fori_loop(0, NQ, one_pass, jnp.zeros((L,), jnp.float32))
        dots_v[gsl] = acc
        return 0

    lax.fori_loop(0, NG, group, 0)

    pltpu.sync_copy(dots_v, out_hbm.at[pl.ds(base, BPW)])


def _sc_dots(w1, w2, W1t3, W2t3):
    mesh = plsc.VectorSubcoreMesh(core_axis_name="c", subcore_axis_name="s")
    return pl.kernel(
        _sc_body,
        out_type=jax.ShapeDtypeStruct((B,), jnp.float32),
        mesh=mesh,
        compiler_params=pltpu.CompilerParams(needs_layout_passes=False),
        scratch_types=[
            pltpu.VMEM((BPW,), jnp.int32),
            pltpu.VMEM((BPW,), jnp.int32),
            pltpu.VMEM((L, AQ, 8, 128), jnp.float32),
            pltpu.VMEM((L, AQ, 8, 128), jnp.float32),
            pltpu.VMEM((BPW,), jnp.float32),
            pltpu.SemaphoreType.DMA,
        ],
    )(w1, w2, W1t3, W2t3)


def _loss_body(dot_ref, lbl_ref, out_ref):
    x = dot_ref[...]
    l = lbl_ref[...]
    sp = jnp.maximum(-x, 0.0) + jnp.log1p(jnp.exp(-jnp.abs(x)))
    out_ref[0, 0] = jnp.sum(l * sp) * (1.0 / B)


def _tc_loss(dots2d, lbl2d):
    return pl.pallas_call(
        _loss_body,
        out_shape=jax.ShapeDtypeStruct((1, 1), jnp.float32),
        out_specs=pl.BlockSpec(memory_space=pltpu.SMEM),
    )(dots2d, lbl2d)


def kernel(word1, word2, label, W1, W2):
    W1t3 = W1.T.reshape(8, 8, VOCAB)
    W2t3 = W2.T.reshape(8, 8, VOCAB)
    dots = _sc_dots(word1.astype(jnp.int32), word2.astype(jnp.int32),
                    W1t3, W2t3)
    loss = _tc_loss(dots.reshape(128, 128),
                    label.astype(jnp.float32).reshape(128, 128))
    return loss[0, 0]
